# baseline (speedup 1.0000x reference)
import functools
import math

import jax
import jax.numpy as jnp
from jax import lax
from jax.experimental import pallas as pl
from jax.experimental.pallas import tpu as pltpu
from jax.experimental.pallas import tpu_sc as plsc

_NC = 2
_NS = 16
_NW = _NC * _NS
_LANES = 16


def _pos_encoding(seq_len, d_embed):
    pos = jnp.arange(seq_len, dtype=jnp.float32)
    denom = jnp.exp(
        -jnp.arange(0, d_embed, 2, dtype=jnp.float32) * math.log(10000.0) / d_embed
    )
    phase = pos[:, None] * denom[None, :]
    enc = jnp.zeros((seq_len, d_embed), dtype=jnp.float32)
    enc = enc.at[:, 0::2].set(jnp.sin(phase))
    enc = enc.at[:, 1::2].set(jnp.cos(phase))
    return enc


def _make_sc_embed(N, T, D, idx_cols):
    chunks_per_worker = N // (_NW * T)
    rows_per_chunk = T
    idx_rows_chunk = rows_per_chunk // idx_cols
    idx_rows_w = N // (_NW * idx_cols)
    scale = float(math.sqrt(D))
    mesh = plsc.VectorSubcoreMesh(core_axis_name="c", subcore_axis_name="s")

    @functools.partial(
        pl.kernel,
        out_type=jax.ShapeDtypeStruct((N, D), jnp.float32),
        mesh=mesh,
        compiler_params=pltpu.CompilerParams(use_tc_tiling_on_sc=False),
        scratch_types=[
            pltpu.VMEM((idx_rows_w, idx_cols), jnp.int32),
            pltpu.VMEM((rows_per_chunk, D), jnp.float32),
            pltpu.VMEM((T, D), jnp.float32),
            pltpu.SemaphoreType.DMA,
        ],
    )
    def k(idx_hbm, table_hbm, pe_hbm, out_hbm, idx_v, rows_v, pe_v, sem):
        wid = lax.axis_index("s") * _NC + lax.axis_index("c")
        pltpu.sync_copy(pe_hbm, pe_v)
        pltpu.sync_copy(idx_hbm.at[pl.ds(wid * idx_rows_w, idx_rows_w)], idx_v)

        def chunk_body(c, carry):
            base = (wid * chunks_per_worker + c) * rows_per_chunk
            cps = [
                pltpu.async_copy(
                    table_hbm.at[idx_v.at[c * idx_rows_chunk + j]],
                    rows_v.at[pl.ds(j * idx_cols, idx_cols)],
                    sem,
                )
                for j in range(idx_rows_chunk)
            ]
            for cp in cps:
                cp.wait()

            def row_body(r, carry2):
                for j in range(D // _LANES):
                    sl = pl.ds(j * _LANES, _LANES)
                    rows_v[r, sl] = rows_v[r, sl] * scale + pe_v[r, sl]
                return carry2

            lax.fori_loop(0, rows_per_chunk, row_body, 0)
            pltpu.sync_copy(rows_v, out_hbm.at[pl.ds(base, rows_per_chunk)])
            return carry

        lax.fori_loop(0, chunks_per_worker, chunk_body, 0)

    return k


def kernel(indices, embed_weight):
    B, T = indices.shape
    V, D = embed_weight.shape
    N = B * T
    idx_cols = 100
    pe = _pos_encoding(T, D)
    idx2d = indices.reshape(N // idx_cols, idx_cols)
    out = _make_sc_embed(N, T, D, idx_cols)(idx2d, embed_weight, pe)
    return out.reshape(B, T, D)


# speedup vs baseline: 1.0528x; 1.0528x over previous
import functools
import math

import jax
import jax.numpy as jnp
from jax import lax
from jax.experimental import pallas as pl
from jax.experimental.pallas import tpu as pltpu
from jax.experimental.pallas import tpu_sc as plsc

_NC = 2
_NS = 16
_NW = _NC * _NS
_LANES = 16


def _pos_encoding(seq_len, d_embed):
    pos = jnp.arange(seq_len, dtype=jnp.float32)
    denom = jnp.exp(
        -jnp.arange(0, d_embed, 2, dtype=jnp.float32) * math.log(10000.0) / d_embed
    )
    phase = pos[:, None] * denom[None, :]
    enc = jnp.zeros((seq_len, d_embed), dtype=jnp.float32)
    enc = enc.at[:, 0::2].set(jnp.sin(phase))
    enc = enc.at[:, 1::2].set(jnp.cos(phase))
    return enc


def _make_sc_embed(N, T, D, idx_cols):
    rows_w = N // _NW
    seqs_mega = 8
    rows_mega = seqs_mega * T
    megas_w = rows_w // rows_mega
    idx_rows_w = rows_w // idx_cols
    idx_rows_mega = rows_mega // idx_cols
    scale = float(math.sqrt(D))
    mesh = plsc.VectorSubcoreMesh(core_axis_name="c", subcore_axis_name="s")

    @functools.partial(
        pl.kernel,
        out_type=jax.ShapeDtypeStruct((N, D), jnp.float32),
        mesh=mesh,
        compiler_params=pltpu.CompilerParams(use_tc_tiling_on_sc=False),
        scratch_types=[
            pltpu.VMEM((idx_rows_w, idx_cols), jnp.int32),
            pltpu.VMEM((rows_mega, D), jnp.float32),
            pltpu.VMEM((T, D), jnp.float32),
            pltpu.SemaphoreType.DMA,
        ],
    )
    def k(idx_hbm, table_hbm, pe_hbm, out_hbm, idx_v, rows_v, pe_v, sem):
        wid = lax.axis_index("s") * _NC + lax.axis_index("c")
        pltpu.sync_copy(pe_hbm, pe_v)
        pltpu.sync_copy(idx_hbm.at[pl.ds(wid * idx_rows_w, idx_rows_w)], idx_v)

        def mega_body(g, carry):
            base = wid * rows_w + g * rows_mega
            cps = [
                pltpu.async_copy(
                    table_hbm.at[idx_v.at[g * idx_rows_mega + j]],
                    rows_v.at[pl.ds(j * idx_cols, idx_cols)],
                    sem,
                )
                for j in range(idx_rows_mega)
            ]
            for cp in cps:
                cp.wait()

            def row_body(r, carry2):
                for j in range(D // _LANES):
                    sl = pl.ds(j * _LANES, _LANES)
                    pe_j = pe_v[r, sl]
                    for s in range(seqs_mega):
                        rows_v[s * T + r, sl] = rows_v[s * T + r, sl] * scale + pe_j
                return carry2

            lax.fori_loop(0, T, row_body, 0)
            pltpu.sync_copy(rows_v, out_hbm.at[pl.ds(base, rows_mega)])
            return carry

        lax.fori_loop(0, megas_w, mega_body, 0)

    return k


def kernel(indices, embed_weight):
    B, T = indices.shape
    V, D = embed_weight.shape
    N = B * T
    idx_cols = 100
    pe = _pos_encoding(T, D)
    idx2d = indices.reshape(N // idx_cols, idx_cols)
    out = _make_sc_embed(N, T, D, idx_cols)(idx2d, embed_weight, pe)
    return out.reshape(B, T, D)
